# baseline (speedup 1.0000x reference)
import functools

import jax
import jax.numpy as jnp
from jax import lax
from jax.experimental import pallas as pl
from jax.experimental.pallas import tpu as pltpu
from jax.experimental.pallas import tpu_sc as plsc

_BATCH = 4096
_HIST = 200
_D = 128
_V = 200
_TOT = _BATCH * _HIST

_info = plsc.get_sparse_core_info()
_NC = _info.num_cores
_NS = _info.num_subcores
_NW = _NC * _NS
_B_W = _TOT // _NW
_CH = 128
_N_CH = _B_W // _CH
_NB = 4
_R = _N_CH // _NB

_mesh = plsc.VectorSubcoreMesh(core_axis_name="c", subcore_axis_name="s")


@functools.partial(
    pl.kernel,
    mesh=_mesh,
    out_type=jax.ShapeDtypeStruct((_TOT, _D), jnp.float32),
    scratch_types=[
        pltpu.VMEM((_N_CH, _CH), jnp.int32),
        pltpu.VMEM_SHARED((_V, _D), jnp.float32),
        pltpu.VMEM((_NB, _CH, _D), jnp.float32),
    ]
    + [pltpu.SemaphoreType.DMA] * (2 * _NB),
)
def _gather_kernel(idx_hbm, table_hbm, out_hbm, idx_v, table_sh, rows, *sems):
    gsem = sems[:_NB]
    ssem = sems[_NB:]
    sid = lax.axis_index("s")
    wid = sid * _NC + lax.axis_index("c")
    base = wid * _B_W

    @pl.when(sid == 0)
    def _stage_table():
        pltpu.sync_copy(table_hbm, table_sh)

    pltpu.sync_copy(idx_hbm.at[pl.ds(wid * _N_CH, _N_CH)], idx_v)
    plsc.subcore_barrier()

    def fire_gather(b, g):
        src = table_sh if b % 2 == 0 else table_hbm
        pltpu.async_copy(src.at[idx_v.at[g]], rows.at[b], gsem[b])

    def wait_gather(b):
        pltpu.make_async_copy(table_sh.at[idx_v.at[0]], rows.at[b], gsem[b]).wait()

    def fire_store(b, g):
        pltpu.async_copy(rows.at[b], out_hbm.at[pl.ds(base + g * _CH, _CH)], ssem[b])

    def wait_store(b):
        pltpu.make_async_copy(rows.at[b], out_hbm.at[pl.ds(base, _CH)], ssem[b]).wait()

    for b in range(_NB):
        fire_gather(b, b)

    def body(r, carry):
        g0 = r * _NB
        for b in range(_NB):
            wait_gather(b)
            fire_store(b, g0 + b)

        @pl.when(r < _R - 1)
        def _():
            for b in range(_NB):
                wait_store(b)
                fire_gather(b, g0 + _NB + b)

        return carry

    lax.fori_loop(0, _R, body, 0)
    for b in range(_NB):
        wait_store(b)


def kernel(t, table):
    out = _gather_kernel(t.reshape(_NW * _N_CH, _CH), table)
    return out.reshape(_BATCH, _HIST, _D)


# speedup vs baseline: 2.3785x; 2.3785x over previous
import functools

import jax
import jax.numpy as jnp
from jax import lax
from jax.experimental import pallas as pl
from jax.experimental.pallas import tpu as pltpu
from jax.experimental.pallas import tpu_sc as plsc

_BATCH = 4096
_HIST = 200
_D = 128
_V = 200
_TOT = _BATCH * _HIST

_info = plsc.get_sparse_core_info()
_NC = _info.num_cores
_NS = _info.num_subcores
_NW = _NC * _NS
_B_W = _TOT // _NW
_CH = 128
_N_CH = _B_W // _CH
_NB = 2
_R = _N_CH // (2 * _NB)

_mesh = plsc.VectorSubcoreMesh(core_axis_name="c", subcore_axis_name="s")


@functools.partial(
    pl.kernel,
    mesh=_mesh,
    out_type=jax.ShapeDtypeStruct((_TOT // 128, 128, _D), jnp.float32),
    scratch_types=[
        pltpu.VMEM((_N_CH, _CH), jnp.int32),
        pltpu.VMEM_SHARED((_V, _D), jnp.float32),
        pltpu.VMEM((_NB, 2, _CH, _D), jnp.float32),
    ]
    + [pltpu.SemaphoreType.DMA] * (2 * _NB),
)
def _gather_kernel(idx_hbm, table_hbm, out_hbm, idx_v, table_sh, rows, *sems):
    gsem = sems[:_NB]
    ssem = sems[_NB:]
    sid = lax.axis_index("s")
    wid = sid * _NC + lax.axis_index("c")
    base = wid * (_B_W // 128)

    @pl.when(sid == 0)
    def _stage_table():
        pltpu.sync_copy(table_hbm, table_sh)

    pltpu.sync_copy(idx_hbm.at[pl.ds(wid * _N_CH, _N_CH)], idx_v)
    plsc.subcore_barrier()

    def fire_gathers(b, c):
        for h in range(2):
            pltpu.async_copy(
                table_sh.at[idx_v.at[2 * c + h]], rows.at[b].at[h], gsem[b]
            )

    def wait_gathers(b):
        pltpu.make_async_copy(
            table_sh.at[idx_v.at[0]], rows.at[b].at[0], gsem[b]
        ).wait()
        pltpu.make_async_copy(
            table_sh.at[idx_v.at[0]], rows.at[b].at[1], gsem[b]
        ).wait()

    def fire_store(b, c):
        pltpu.async_copy(rows.at[b], out_hbm.at[pl.ds(base + c * 2, 2)], ssem[b])

    def wait_store(b):
        pltpu.make_async_copy(rows.at[b], out_hbm.at[pl.ds(base, 2)], ssem[b]).wait()

    for b in range(_NB):
        fire_gathers(b, b)

    def body(r, carry):
        c0 = r * _NB
        for b in range(_NB):
            wait_gathers(b)
            fire_store(b, c0 + b)

        @pl.when(r < _R - 1)
        def _():
            for b in range(_NB):
                wait_store(b)
                fire_gathers(b, c0 + _NB + b)

        return carry

    lax.fori_loop(0, _R, body, 0)
    for b in range(_NB):
        wait_store(b)


def kernel(t, table):
    out = _gather_kernel(t.reshape(_NW * _N_CH, _CH), table)
    return out.reshape(_BATCH, _HIST, _D)


# speedup vs baseline: 3.4589x; 1.4542x over previous
import functools

import jax
import jax.numpy as jnp
from jax import lax
from jax.experimental import pallas as pl
from jax.experimental.pallas import tpu as pltpu
from jax.experimental.pallas import tpu_sc as plsc

_BATCH = 4096
_HIST = 200
_D = 128
_V = 200
_TOT = _BATCH * _HIST

_info = plsc.get_sparse_core_info()
_NC = _info.num_cores
_NS = _info.num_subcores
_NW = _NC * _NS
_B_W = _TOT // _NW
_CH = 64
_N_CH = _B_W // _CH
_NB = 8
_R = _N_CH // _NB

_mesh = plsc.VectorSubcoreMesh(core_axis_name="c", subcore_axis_name="s")


@functools.partial(
    pl.kernel,
    mesh=_mesh,
    out_type=jax.ShapeDtypeStruct((_TOT, _D), jnp.float32),
    scratch_types=[
        pltpu.VMEM((_N_CH, _CH), jnp.int32),
        pltpu.VMEM_SHARED((_V, _D), jnp.float32),
        pltpu.VMEM((_NB, _CH, _D), jnp.float32),
    ]
    + [pltpu.SemaphoreType.DMA] * (2 * _NB),
)
def _gather_kernel(idx_hbm, table_hbm, out_hbm, idx_v, table_sh, rows, *sems):
    gsem = sems[:_NB]
    ssem = sems[_NB:]
    sid = lax.axis_index("s")
    wid = sid * _NC + lax.axis_index("c")
    base = wid * _B_W

    @pl.when(sid == 0)
    def _stage_table():
        pltpu.sync_copy(table_hbm, table_sh)

    pltpu.sync_copy(idx_hbm.at[pl.ds(wid * _N_CH, _N_CH)], idx_v)
    plsc.subcore_barrier()

    def fire_gather(b, g):
        pltpu.async_copy(table_sh.at[idx_v.at[g]], rows.at[b], gsem[b])

    def wait_gather(b):
        pltpu.make_async_copy(table_sh.at[idx_v.at[0]], rows.at[b], gsem[b]).wait()

    def fire_store(b, g):
        pltpu.async_copy(rows.at[b], out_hbm.at[pl.ds(base + g * _CH, _CH)], ssem[b])

    def wait_store(b):
        pltpu.make_async_copy(rows.at[b], out_hbm.at[pl.ds(base, _CH)], ssem[b]).wait()

    for b in range(_NB):
        fire_gather(b, b)

    def body(r, carry):
        g0 = r * _NB
        for b in range(_NB):
            wait_gather(b)
            fire_store(b, g0 + b)

        @pl.when(r < _R - 1)
        def _():
            for b in range(_NB):
                wait_store(b)
                fire_gather(b, g0 + _NB + b)

        return carry

    lax.fori_loop(0, _R, body, 0)
    for b in range(_NB):
        wait_store(b)


def kernel(t, table):
    out = _gather_kernel(t.reshape(_NW * _N_CH, _CH), table)
    return out.reshape(_BATCH, _HIST, _D)


# speedup vs baseline: 3.4945x; 1.0103x over previous
import functools

import jax
import jax.numpy as jnp
from jax import lax
from jax.experimental import pallas as pl
from jax.experimental.pallas import tpu as pltpu
from jax.experimental.pallas import tpu_sc as plsc

_BATCH = 4096
_HIST = 200
_D = 128
_V = 200
_TOT = _BATCH * _HIST

_info = plsc.get_sparse_core_info()
_NC = _info.num_cores
_NS = _info.num_subcores
_NW = _NC * _NS
_B_W = _TOT // _NW
_CH = 64
_N_CH = _B_W // _CH
_NB = 8
_R = _N_CH // _NB

_mesh = plsc.VectorSubcoreMesh(core_axis_name="c", subcore_axis_name="s")


@functools.partial(
    pl.kernel,
    mesh=_mesh,
    out_type=jax.ShapeDtypeStruct((_TOT, _D), jnp.float32),
    scratch_types=[
        pltpu.VMEM((_N_CH, _CH), jnp.int32),
        pltpu.VMEM_SHARED((_V, _D), jnp.float32),
        pltpu.VMEM((_NB, _CH, _D), jnp.float32),
    ]
    + [pltpu.SemaphoreType.DMA] * (2 * _NB + 1),
)
def _gather_kernel(idx_hbm, table_hbm, out_hbm, idx_v, table_sh, rows, *sems):
    gsem = sems[:_NB]
    ssem = sems[_NB : 2 * _NB]
    isem = sems[2 * _NB]
    sid = lax.axis_index("s")
    wid = sid * _NC + lax.axis_index("c")
    base = wid * _B_W

    idx_copy = pltpu.async_copy(idx_hbm.at[pl.ds(wid * _N_CH, _N_CH)], idx_v, isem)

    @pl.when(sid == 0)
    def _stage_table():
        pltpu.sync_copy(table_hbm, table_sh)

    plsc.subcore_barrier()
    idx_copy.wait()

    def fire_gather(b, g):
        pltpu.async_copy(table_sh.at[idx_v.at[g]], rows.at[b], gsem[b])

    def wait_gather(b):
        pltpu.make_async_copy(table_sh.at[idx_v.at[0]], rows.at[b], gsem[b]).wait()

    def fire_store(b, g):
        pltpu.async_copy(rows.at[b], out_hbm.at[pl.ds(base + g * _CH, _CH)], ssem[b])

    def wait_store(b):
        pltpu.make_async_copy(rows.at[b], out_hbm.at[pl.ds(base, _CH)], ssem[b]).wait()

    for b in range(_NB):
        fire_gather(b, b)

    def body(r, carry):
        g0 = r * _NB
        for b in range(_NB):
            wait_gather(b)
            fire_store(b, g0 + b)

        @pl.when(r < _R - 1)
        def _():
            for b in range(_NB):
                wait_store(b)
                fire_gather(b, g0 + _NB + b)

        return carry

    lax.fori_loop(0, _R, body, 0)
    for b in range(_NB):
        wait_store(b)


def kernel(t, table):
    out = _gather_kernel(t.reshape(_NW * _N_CH, _CH), table)
    return out.reshape(_BATCH, _HIST, _D)
